# baseline (speedup 1.0000x reference)
import functools

import jax
import jax.numpy as jnp
from jax import lax
from jax.experimental import pallas as pl
from jax.experimental.pallas import tpu as pltpu
from jax.experimental.pallas import tpu_sc as plsc

_JB = 1000


def _fold_body(x_ref, xtb_ref, v_out, b_out):
    j = pl.program_id(0)

    @pl.when(j == 0)
    def _init():
        v_out[...] = jnp.full_like(v_out, jnp.inf)
        b_out[...] = jnp.zeros_like(b_out)

    x = x_ref[...]
    xtb = xtb_ref[0]
    s = lax.dot_general(
        x, xtb, (((1,), (0,)), ((), ())),
        preferred_element_type=jnp.float32,
        precision=lax.Precision.DEFAULT,
    )
    a2 = jnp.sum(x * x, axis=1, keepdims=True)
    b2 = 0.25 * jnp.sum(xtb * xtb, axis=0)
    d2 = (a2 + s) + b2[None, :]

    rv = v_out[...]
    better = d2 < rv
    v_out[...] = jnp.where(better, d2, rv)
    b_out[...] = jnp.where(better, j, b_out[...])


def _extract_body(v_ref, b_ref, idx_ref):
    rv = v_ref[...]
    rb = b_ref[...]
    gmin = jnp.min(rv, axis=1, keepdims=True)
    lane = lax.broadcasted_iota(jnp.int32, rv.shape, 1)
    cand = jnp.where(rv == gmin, rb * _JB + lane, jnp.int32(2**30))
    idx_ref[...] = jnp.min(cand, axis=1, keepdims=True)


def _nearest_idx(x_flat, X_train):
    b, k = x_flat.shape
    n = X_train.shape[0]
    nj = n // _JB
    xt3 = (-2.0 * X_train.T).reshape(k, nj, _JB).transpose(1, 0, 2)
    run_v, run_b = pl.pallas_call(
        _fold_body,
        grid=(nj,),
        in_specs=[
            pl.BlockSpec((b, k), lambda j: (0, 0)),
            pl.BlockSpec((1, k, _JB), lambda j: (j, 0, 0)),
        ],
        out_specs=[
            pl.BlockSpec((b, _JB), lambda j: (0, 0)),
            pl.BlockSpec((b, _JB), lambda j: (0, 0)),
        ],
        out_shape=[
            jax.ShapeDtypeStruct((b, _JB), jnp.float32),
            jax.ShapeDtypeStruct((b, _JB), jnp.int32),
        ],
    )(x_flat, xt3)

    rb_rows = 128
    return pl.pallas_call(
        _extract_body,
        grid=(b // rb_rows,),
        in_specs=[
            pl.BlockSpec((rb_rows, _JB), lambda i: (i, 0)),
            pl.BlockSpec((rb_rows, _JB), lambda i: (i, 0)),
        ],
        out_specs=pl.BlockSpec((rb_rows, 1), lambda i: (i, 0)),
        out_shape=jax.ShapeDtypeStruct((b, 1), jnp.int32),
    )(run_v, run_b)


def _gather_body(bpw, y_hbm, idx_hbm, out_hbm, idx_v, rows_v, sem):
    wid = lax.axis_index("s") * 2 + lax.axis_index("c")
    base = wid * bpw
    pltpu.sync_copy(idx_hbm.at[pl.ds(base, bpw)], idx_v)
    pltpu.async_copy(y_hbm.at[idx_v], rows_v, sem).wait()
    pltpu.sync_copy(rows_v, out_hbm.at[pl.ds(base, bpw)])


def _gather_rows(Y2d, idx):
    b = idx.shape[0]
    d = Y2d.shape[1]
    nw = 32
    bpw = b // nw
    mesh = plsc.VectorSubcoreMesh(core_axis_name="c", subcore_axis_name="s")
    return pl.kernel(
        functools.partial(_gather_body, bpw),
        out_type=jax.ShapeDtypeStruct((b, d), jnp.float32),
        mesh=mesh,
        compiler_params=pltpu.CompilerParams(use_tc_tiling_on_sc=False),
        scratch_types=[
            pltpu.VMEM((bpw,), jnp.int32),
            pltpu.VMEM((bpw, d), jnp.float32),
            pltpu.SemaphoreType.DMA,
        ],
    )(Y2d, idx)


def kernel(x, X_train, Y_train):
    b = x.shape[0]
    x_flat = x.reshape(b, -1)
    idx = _nearest_idx(x_flat, X_train)
    n, dy = Y_train.shape[0], Y_train.shape[1]
    y = _gather_rows(Y_train.reshape(n, dy), idx.reshape(b))
    return y.reshape(b, dy, 1)


# speedup vs baseline: 1.0699x; 1.0699x over previous
import functools

import jax
import jax.numpy as jnp
from jax import lax
from jax.experimental import pallas as pl
from jax.experimental.pallas import tpu as pltpu
from jax.experimental.pallas import tpu_sc as plsc

_JB = 1000


def _fold_body(x_ref, xtb_ref, v_out, b_out):
    j = pl.program_id(0)

    @pl.when(j == 0)
    def _init():
        v_out[...] = jnp.full_like(v_out, jnp.inf)
        b_out[...] = jnp.zeros_like(b_out)

    xs = x_ref[...]
    xtb = xtb_ref[0]
    s = lax.dot_general(
        xs, xtb, (((1,), (0,)), ((), ())),
        preferred_element_type=jnp.float32,
        precision=lax.Precision.DEFAULT,
    )
    a2 = 0.25 * jnp.sum(xs * xs, axis=1, keepdims=True)
    b2 = jnp.sum(xtb * xtb, axis=0)
    d2 = (a2 + s) + b2[None, :]

    rv = v_out[...]
    better = d2 < rv
    v_out[...] = jnp.where(better, d2, rv)
    b_out[...] = jnp.where(better, j, b_out[...])


def _extract_body(v_ref, b_ref, idx_ref):
    rv = v_ref[...]
    rb = b_ref[...]
    gmin = jnp.min(rv, axis=1, keepdims=True)
    lane = lax.broadcasted_iota(jnp.int32, rv.shape, 1)
    cand = jnp.where(rv == gmin, rb * _JB + lane, jnp.int32(2**30))
    idx_ref[...] = jnp.min(cand, axis=1, keepdims=True)


def _nearest_idx(x_flat, X_train):
    b, k = x_flat.shape
    n = X_train.shape[0]
    nj = n // _JB
    xt3 = X_train.T.reshape(k, nj, _JB).transpose(1, 0, 2)
    run_v, run_b = pl.pallas_call(
        _fold_body,
        grid=(nj,),
        in_specs=[
            pl.BlockSpec((b, k), lambda j: (0, 0)),
            pl.BlockSpec((1, k, _JB), lambda j: (j, 0, 0)),
        ],
        out_specs=[
            pl.BlockSpec((b, _JB), lambda j: (0, 0)),
            pl.BlockSpec((b, _JB), lambda j: (0, 0)),
        ],
        out_shape=[
            jax.ShapeDtypeStruct((b, _JB), jnp.float32),
            jax.ShapeDtypeStruct((b, _JB), jnp.int32),
        ],
    )(-2.0 * x_flat, xt3)

    rb_rows = 128
    return pl.pallas_call(
        _extract_body,
        grid=(b // rb_rows,),
        in_specs=[
            pl.BlockSpec((rb_rows, _JB), lambda i: (i, 0)),
            pl.BlockSpec((rb_rows, _JB), lambda i: (i, 0)),
        ],
        out_specs=pl.BlockSpec((rb_rows, 1), lambda i: (i, 0)),
        out_shape=jax.ShapeDtypeStruct((b, 1), jnp.int32),
    )(run_v, run_b)


def _gather_body(bpw, y_hbm, idx_hbm, out_hbm, idx_v, rows_v, sem):
    wid = lax.axis_index("s") * 2 + lax.axis_index("c")
    base = wid * bpw
    pltpu.sync_copy(idx_hbm.at[pl.ds(base, bpw)], idx_v)
    pltpu.async_copy(y_hbm.at[idx_v], rows_v, sem).wait()
    pltpu.sync_copy(rows_v, out_hbm.at[pl.ds(base, bpw)])


def _gather_rows(Y2d, idx):
    b = idx.shape[0]
    d = Y2d.shape[1]
    nw = 32
    bpw = b // nw
    mesh = plsc.VectorSubcoreMesh(core_axis_name="c", subcore_axis_name="s")
    return pl.kernel(
        functools.partial(_gather_body, bpw),
        out_type=jax.ShapeDtypeStruct((b, d), jnp.float32),
        mesh=mesh,
        compiler_params=pltpu.CompilerParams(use_tc_tiling_on_sc=False),
        scratch_types=[
            pltpu.VMEM((bpw,), jnp.int32),
            pltpu.VMEM((bpw, d), jnp.float32),
            pltpu.SemaphoreType.DMA,
        ],
    )(Y2d, idx)


def kernel(x, X_train, Y_train):
    b = x.shape[0]
    x_flat = x.reshape(b, -1)
    idx = _nearest_idx(x_flat, X_train)
    n, dy = Y_train.shape[0], Y_train.shape[1]
    y = _gather_rows(Y_train.reshape(n, dy), idx.reshape(b))
    return y.reshape(b, dy, 1)


# speedup vs baseline: 1.0807x; 1.0100x over previous
import functools

import jax
import jax.numpy as jnp
from jax import lax
from jax.experimental import pallas as pl
from jax.experimental.pallas import tpu as pltpu
from jax.experimental.pallas import tpu_sc as plsc

_JB = 1000


def _fold_body(x_ref, xtb_ref, v_out, b_out):
    j = pl.program_id(0)

    @pl.when(j == 0)
    def _init():
        v_out[...] = jnp.full_like(v_out, jnp.inf)
        b_out[...] = jnp.zeros_like(b_out)

    xs = x_ref[...]
    a2 = 0.25 * jnp.sum(xs * xs, axis=1, keepdims=True)

    def _d2(t):
        xtb = xtb_ref[t]
        s = lax.dot_general(
            xs, xtb, (((1,), (0,)), ((), ())),
            preferred_element_type=jnp.float32,
            precision=lax.Precision.DEFAULT,
        )
        b2 = jnp.sum(xtb * xtb, axis=0)
        return (a2 + s) + b2[None, :]

    d2a = _d2(0)
    d2b = _d2(1)
    bwins = d2b < d2a
    dmin = jnp.minimum(d2a, d2b)
    jmin = jnp.where(bwins, 2 * j + 1, 2 * j)

    rv = v_out[...]
    better = dmin < rv
    v_out[...] = jnp.where(better, dmin, rv)
    b_out[...] = jnp.where(better, jmin, b_out[...])


def _extract_body(v_ref, b_ref, idx_ref):
    rv = v_ref[...]
    rb = b_ref[...]
    gmin = jnp.min(rv, axis=1, keepdims=True)
    lane = lax.broadcasted_iota(jnp.int32, rv.shape, 1)
    cand = jnp.where(rv == gmin, rb * _JB + lane, jnp.int32(2**30))
    idx_ref[...] = jnp.min(cand, axis=1, keepdims=True)


def _nearest_idx(x_flat, X_train):
    b, k = x_flat.shape
    n = X_train.shape[0]
    nj = n // _JB
    xt3 = X_train.T.reshape(k, nj, _JB).transpose(1, 0, 2)
    run_v, run_b = pl.pallas_call(
        _fold_body,
        grid=(nj // 2,),
        in_specs=[
            pl.BlockSpec((b, k), lambda j: (0, 0)),
            pl.BlockSpec((2, k, _JB), lambda j: (j, 0, 0)),
        ],
        out_specs=[
            pl.BlockSpec((b, _JB), lambda j: (0, 0)),
            pl.BlockSpec((b, _JB), lambda j: (0, 0)),
        ],
        out_shape=[
            jax.ShapeDtypeStruct((b, _JB), jnp.float32),
            jax.ShapeDtypeStruct((b, _JB), jnp.int32),
        ],
    )(-2.0 * x_flat, xt3)

    rb_rows = 128
    return pl.pallas_call(
        _extract_body,
        grid=(b // rb_rows,),
        in_specs=[
            pl.BlockSpec((rb_rows, _JB), lambda i: (i, 0)),
            pl.BlockSpec((rb_rows, _JB), lambda i: (i, 0)),
        ],
        out_specs=pl.BlockSpec((rb_rows, 1), lambda i: (i, 0)),
        out_shape=jax.ShapeDtypeStruct((b, 1), jnp.int32),
    )(run_v, run_b)


def _gather_body(bpw, y_hbm, idx_hbm, out_hbm, idx_v, rows_v, sem):
    wid = lax.axis_index("s") * 2 + lax.axis_index("c")
    base = wid * bpw
    pltpu.sync_copy(idx_hbm.at[pl.ds(base, bpw)], idx_v)
    pltpu.async_copy(y_hbm.at[idx_v], rows_v, sem).wait()
    pltpu.sync_copy(rows_v, out_hbm.at[pl.ds(base, bpw)])


def _gather_rows(Y2d, idx):
    b = idx.shape[0]
    d = Y2d.shape[1]
    nw = 32
    bpw = b // nw
    mesh = plsc.VectorSubcoreMesh(core_axis_name="c", subcore_axis_name="s")
    return pl.kernel(
        functools.partial(_gather_body, bpw),
        out_type=jax.ShapeDtypeStruct((b, d), jnp.float32),
        mesh=mesh,
        compiler_params=pltpu.CompilerParams(use_tc_tiling_on_sc=False),
        scratch_types=[
            pltpu.VMEM((bpw,), jnp.int32),
            pltpu.VMEM((bpw, d), jnp.float32),
            pltpu.SemaphoreType.DMA,
        ],
    )(Y2d, idx)


def kernel(x, X_train, Y_train):
    b = x.shape[0]
    x_flat = x.reshape(b, -1)
    idx = _nearest_idx(x_flat, X_train)
    n, dy = Y_train.shape[0], Y_train.shape[1]
    y = _gather_rows(Y_train.reshape(n, dy), idx.reshape(b))
    return y.reshape(b, dy, 1)


# speedup vs baseline: 1.1416x; 1.0564x over previous
import functools

import jax
import jax.numpy as jnp
from jax import lax
from jax.experimental import pallas as pl
from jax.experimental.pallas import tpu as pltpu
from jax.experimental.pallas import tpu_sc as plsc

_JB = 1000


def _fold_body(xst_ref, xb_ref, v_out, b_out):
    j = pl.program_id(0)

    @pl.when(j == 0)
    def _init():
        v_out[...] = jnp.full_like(v_out, jnp.inf)
        b_out[...] = jnp.zeros_like(b_out)

    xst = xst_ref[...]
    xb = xb_ref[0]
    s = lax.dot_general(
        xb, xst, (((1,), (0,)), ((), ())),
        preferred_element_type=jnp.float32,
        precision=lax.Precision.DEFAULT,
    )
    a2 = 0.25 * jnp.sum(xst * xst, axis=0)
    b2 = jnp.sum(xb * xb, axis=1, keepdims=True)
    d2 = (a2[None, :] + s) + b2

    rv = v_out[...]
    better = d2 < rv
    v_out[...] = jnp.where(better, d2, rv)
    b_out[...] = jnp.where(better, j, b_out[...])


def _extract_body(v_ref, b_ref, idx_ref):
    rv = v_ref[...]
    rb = b_ref[...]
    gmin = jnp.min(rv, axis=0, keepdims=True)
    row = lax.broadcasted_iota(jnp.int32, rv.shape, 0)
    cand = jnp.where(rv == gmin, rb * _JB + row, jnp.int32(2**30))
    idx_ref[...] = jnp.min(cand, axis=0, keepdims=True)


def _nearest_idx(x_flat, X_train):
    b, k = x_flat.shape
    n = X_train.shape[0]
    nj = n // _JB
    x3 = X_train.reshape(nj, _JB, k)
    xst = (-2.0 * x_flat).T
    run_v, run_b = pl.pallas_call(
        _fold_body,
        grid=(nj,),
        in_specs=[
            pl.BlockSpec((k, b), lambda j: (0, 0)),
            pl.BlockSpec((1, _JB, k), lambda j: (j, 0, 0)),
        ],
        out_specs=[
            pl.BlockSpec((_JB, b), lambda j: (0, 0)),
            pl.BlockSpec((_JB, b), lambda j: (0, 0)),
        ],
        out_shape=[
            jax.ShapeDtypeStruct((_JB, b), jnp.float32),
            jax.ShapeDtypeStruct((_JB, b), jnp.int32),
        ],
    )(xst, x3)

    return pl.pallas_call(
        _extract_body,
        grid=(1,),
        in_specs=[
            pl.BlockSpec((_JB, b), lambda i: (0, 0)),
            pl.BlockSpec((_JB, b), lambda i: (0, 0)),
        ],
        out_specs=pl.BlockSpec((1, b), lambda i: (0, 0)),
        out_shape=jax.ShapeDtypeStruct((1, b), jnp.int32),
    )(run_v, run_b)


def _gather_body(bpw, y_hbm, idx_hbm, out_hbm, idx_v, rows_v, sem):
    wid = lax.axis_index("s") * 2 + lax.axis_index("c")
    base = wid * bpw
    pltpu.sync_copy(idx_hbm.at[pl.ds(base, bpw)], idx_v)
    pltpu.async_copy(y_hbm.at[idx_v], rows_v, sem).wait()
    pltpu.sync_copy(rows_v, out_hbm.at[pl.ds(base, bpw)])


def _gather_rows(Y2d, idx):
    b = idx.shape[0]
    d = Y2d.shape[1]
    nw = 32
    bpw = b // nw
    mesh = plsc.VectorSubcoreMesh(core_axis_name="c", subcore_axis_name="s")
    return pl.kernel(
        functools.partial(_gather_body, bpw),
        out_type=jax.ShapeDtypeStruct((b, d), jnp.float32),
        mesh=mesh,
        compiler_params=pltpu.CompilerParams(use_tc_tiling_on_sc=False),
        scratch_types=[
            pltpu.VMEM((bpw,), jnp.int32),
            pltpu.VMEM((bpw, d), jnp.float32),
            pltpu.SemaphoreType.DMA,
        ],
    )(Y2d, idx)


def kernel(x, X_train, Y_train):
    b = x.shape[0]
    x_flat = x.reshape(b, -1)
    idx = _nearest_idx(x_flat, X_train)
    n, dy = Y_train.shape[0], Y_train.shape[1]
    y = _gather_rows(Y_train.reshape(n, dy), idx.reshape(b))
    return y.reshape(b, dy, 1)
